# Initial kernel scaffold; baseline (speedup 1.0000x reference)
import jax
import jax.numpy as jnp
from jax.experimental import pallas as pl


def kernel(nf1, nf2, nf3, nf4, disjoint, nf3_neg, class_emb, rel_emb):
    raise NotImplementedError("write your pallas kernel here")


# speedup vs baseline: 1.2378x; 1.2378x over previous
import functools

import jax
import jax.numpy as jnp
from jax import lax
from jax.experimental import pallas as pl
from jax.experimental.pallas import tpu as pltpu
from jax.experimental.pallas import tpu_sc as plsc

_EMB = 128
_BATCH = 512
_NWORKERS = 32
_RPW = _BATCH // _NWORKERS
_NCLS = 13
_NREL = 3
_PROWS = 4 * _RPW + 4


def _sc_stage():
    mesh = plsc.VectorSubcoreMesh(core_axis_name="c", subcore_axis_name="s")

    @functools.partial(
        pl.kernel,
        out_type=jax.ShapeDtypeStruct((_NWORKERS, _PROWS, 16), jnp.float32),
        mesh=mesh,
        scratch_types=[
            pltpu.VMEM((16, _RPW), jnp.int32),
            pltpu.VMEM((_NCLS, _RPW, 2 * _EMB), jnp.float32),
            pltpu.VMEM((_NREL, _RPW, _EMB), jnp.float32),
            pltpu.VMEM((_PROWS, 16), jnp.float32),
            pltpu.SemaphoreType.DMA,
        ],
    )
    def sc_k(idx_hbm, cls_hbm, rel_hbm, out_hbm, idxv, cbuf, rbuf, sbuf, sem):
        wid = lax.axis_index("s") * 2 + lax.axis_index("c")
        pltpu.sync_copy(idx_hbm.at[wid], idxv)
        descs = []
        for t in range(_NCLS):
            descs.append(pltpu.async_copy(cls_hbm.at[idxv[t, :]], cbuf.at[t], sem))
        for t in range(_NREL):
            descs.append(
                pltpu.async_copy(rel_hbm.at[idxv[_NCLS + t, :]], rbuf.at[t], sem))
        for d in descs:
            d.wait()

        zero = jnp.zeros((16,), jnp.float32)

        def row(r, carry):
            acc1, acc3, acc4 = carry
            sa = zero
            sb = zero
            sd = zero
            sn = zero
            for ch in range(_EMB // 16):
                lo = pl.ds(ch * 16, 16)
                hi = pl.ds(_EMB + ch * 16, 16)
                cc = cbuf[0, r, lo]
                co = jnp.abs(cbuf[0, r, hi])
                dc = cbuf[1, r, lo]
                do = jnp.abs(cbuf[1, r, hi])
                u = jnp.maximum(jnp.abs(cc - dc) + co - do, 0.0)
                acc1 = acc1 + u * u
                cc = cbuf[2, r, lo]
                co = jnp.abs(cbuf[2, r, hi])
                dc = cbuf[3, r, lo]
                do = jnp.abs(cbuf[3, r, hi])
                ec = cbuf[4, r, lo]
                eo = jnp.abs(cbuf[4, r, hi])
                ll = jnp.maximum(cc - co, dc - do)
                ur = jnp.minimum(cc + co, dc + do)
                dlu = ll - ur
                u = jnp.maximum(
                    jnp.abs((ll + ur) * 0.5 - ec) + jnp.abs(dlu) * 0.5 - eo, 0.0)
                sa = sa + u * u
                v = jnp.maximum(dlu, 0.0)
                sb = sb + v * v
                cc = cbuf[5, r, lo]
                co = jnp.abs(cbuf[5, r, hi])
                dc = cbuf[6, r, lo]
                do = jnp.abs(cbuf[6, r, hi])
                rr = rbuf[0, r, lo]
                u = jnp.maximum(jnp.abs(cc + rr - dc) + co - do, 0.0)
                acc3 = acc3 + u * u
                rr = rbuf[1, r, lo]
                cc = cbuf[7, r, lo]
                co = jnp.abs(cbuf[7, r, hi])
                dc = cbuf[8, r, lo]
                do = jnp.abs(cbuf[8, r, hi])
                u = jnp.maximum(jnp.abs(cc - rr - dc) + co - do, 0.0)
                acc4 = acc4 + u * u
                cc = cbuf[9, r, lo]
                co = jnp.abs(cbuf[9, r, hi])
                dc = cbuf[10, r, lo]
                do = jnp.abs(cbuf[10, r, hi])
                u = jnp.maximum(jnp.abs(cc - dc) - co - do, 0.0)
                sd = sd + u * u
                cc = cbuf[11, r, lo]
                co = jnp.abs(cbuf[11, r, hi])
                dc = cbuf[12, r, lo]
                do = jnp.abs(cbuf[12, r, hi])
                rr = rbuf[2, r, lo]
                u = jnp.maximum(jnp.abs(cc + rr - dc) - co - do, 0.0)
                sn = sn + u * u
            sbuf[r, :] = sa
            sbuf[_RPW + r, :] = sb
            sbuf[2 * _RPW + r, :] = sd
            sbuf[3 * _RPW + r, :] = sn
            return acc1, acc3, acc4

        acc1, acc3, acc4 = lax.fori_loop(0, _RPW, row, (zero, zero, zero))
        sbuf[4 * _RPW, :] = acc1
        sbuf[4 * _RPW + 1, :] = acc3
        sbuf[4 * _RPW + 2, :] = acc4
        sbuf[4 * _RPW + 3, :] = zero
        pltpu.sync_copy(sbuf, out_hbm.at[wid])

    return sc_k


def _combine_body(x_ref, o_ref):
    x = x_ref[...]
    inv_b = 1.0 / _BATCH
    sa = jnp.sum(x[:, 0:_RPW, :], axis=2)
    sb = jnp.sum(x[:, _RPW:2 * _RPW, :], axis=2)
    sd = jnp.sum(x[:, 2 * _RPW:3 * _RPW, :], axis=2)
    sn = jnp.sum(x[:, 3 * _RPW:4 * _RPW, :], axis=2)
    p0 = jnp.sum(x[:, 4 * _RPW, :])
    p5 = jnp.sum(x[:, 4 * _RPW + 1, :])
    p6 = jnp.sum(x[:, 4 * _RPW + 2, :])
    a = jnp.sqrt(sa)
    b = jnp.sqrt(sb)
    p1 = jnp.sum(a)
    p2 = jnp.sum(sa)
    p3 = jnp.sum(b)
    p4 = jnp.sum(sb)
    p7 = jnp.sum(jnp.maximum(2.0 - jnp.sqrt(sd), 0.0) ** 2)
    p8 = jnp.sum(jnp.sqrt(sn))
    p9 = jnp.sum(sn)
    loss = (p0 * inv_b
            + p2 * inv_b + 2.0 * (p1 * inv_b) * (p3 * inv_b) + p4 * inv_b
            + p7 * inv_b
            + p5 * inv_b + p6 * inv_b
            + 4.0 - 4.0 * p8 * inv_b + p9 * inv_b)
    o_ref[0, 0] = loss


def _tc_combine(partials):
    return pl.pallas_call(
        _combine_body,
        out_shape=jax.ShapeDtypeStruct((1, 1), jnp.float32),
        in_specs=[pl.BlockSpec(memory_space=pltpu.VMEM)],
        out_specs=pl.BlockSpec(memory_space=pltpu.SMEM),
    )(partials)


def kernel(nf1, nf2, nf3, nf4, disjoint, nf3_neg, class_emb, rel_emb):
    b = _BATCH
    idx_all = jnp.stack([
        nf1[:b, 0], nf1[:b, 1],
        nf2[:b, 0], nf2[:b, 1], nf2[:b, 2],
        nf3[:b, 0], nf3[:b, 2],
        nf4[:b, 1], nf4[:b, 2],
        disjoint[:b, 0], disjoint[:b, 1],
        nf3_neg[:b, 0], nf3_neg[:b, 2],
        nf3[:b, 1], nf4[:b, 0], nf3_neg[:b, 1],
    ])
    idx_w = idx_all.reshape(16, _NWORKERS, _RPW).transpose(1, 0, 2)
    partials = _sc_stage()(idx_w, class_emb, rel_emb)
    return _tc_combine(partials)[0, 0]


# speedup vs baseline: 1.2639x; 1.0211x over previous
import functools

import jax
import jax.numpy as jnp
from jax import lax
from jax.experimental import pallas as pl
from jax.experimental.pallas import tpu as pltpu
from jax.experimental.pallas import tpu_sc as plsc

_EMB = 128
_BATCH = 512
_NWORKERS = 32
_RPW = _BATCH // _NWORKERS
_NCLS = 13
_NREL = 3
_PROWS = 4 * _RPW + 4
_NCHUNK = _EMB // 16


def _sc_stage():
    mesh = plsc.VectorSubcoreMesh(core_axis_name="c", subcore_axis_name="s")

    @functools.partial(
        pl.kernel,
        out_type=jax.ShapeDtypeStruct((_NWORKERS, _PROWS, 16), jnp.float32),
        mesh=mesh,
        scratch_types=[
            pltpu.VMEM((16, _RPW), jnp.int32),
            pltpu.VMEM((_NCLS, _RPW, 2 * _EMB), jnp.float32),
            pltpu.VMEM((_NREL, _RPW, _EMB), jnp.float32),
            pltpu.VMEM((_PROWS, 16), jnp.float32),
            [pltpu.SemaphoreType.DMA] * 6,
        ],
    )
    def sc_k(idx_hbm, cls_hbm, rel_hbm, out_hbm, idxv, cbuf, rbuf, sbuf, sems):
        wid = lax.axis_index("s") * 2 + lax.axis_index("c")
        pltpu.sync_copy(idx_hbm.at[wid], idxv)

        def cgather(t, dst, g):
            return pltpu.async_copy(cls_hbm.at[idxv[t, :]], cbuf.at[dst], sems[g])

        def rgather(t, dst, g):
            return pltpu.async_copy(rel_hbm.at[idxv[t, :]], rbuf.at[dst], sems[g])

        groups = [
            [cgather(0, 0, 0), cgather(1, 1, 0)],
            [cgather(2, 2, 1), cgather(3, 3, 1), cgather(4, 4, 1)],
            [cgather(5, 5, 2), cgather(7, 6, 2), rgather(6, 0, 2)],
            [cgather(9, 7, 3), cgather(10, 8, 3), rgather(8, 1, 3)],
            [cgather(11, 9, 4), cgather(12, 10, 4)],
            [cgather(13, 11, 5), cgather(15, 12, 5), rgather(14, 2, 5)],
        ]

        zero = jnp.zeros((16,), jnp.float32)

        def halves(buf, t, r, ch):
            c = buf[t, r, pl.ds(ch * 16, 16)]
            o = jnp.abs(buf[t, r, pl.ds(_EMB + ch * 16, 16)])
            return c, o

        for d in groups[0]:
            d.wait()

        def row1(r, acc):
            for ch in range(_NCHUNK):
                cc, co = halves(cbuf, 0, r, ch)
                dc, do = halves(cbuf, 1, r, ch)
                u = jnp.maximum(jnp.abs(cc - dc) + co - do, 0.0)
                acc = acc + u * u
            return acc

        acc1 = lax.fori_loop(0, _RPW, row1, zero)

        for d in groups[1]:
            d.wait()

        def row2(r, _):
            sa = zero
            sb = zero
            for ch in range(_NCHUNK):
                cc, co = halves(cbuf, 2, r, ch)
                dc, do = halves(cbuf, 3, r, ch)
                ec, eo = halves(cbuf, 4, r, ch)
                ll = jnp.maximum(cc - co, dc - do)
                ur = jnp.minimum(cc + co, dc + do)
                dlu = ll - ur
                u = jnp.maximum(
                    jnp.abs((ll + ur) * 0.5 - ec) + jnp.abs(dlu) * 0.5 - eo, 0.0)
                sa = sa + u * u
                v = jnp.maximum(dlu, 0.0)
                sb = sb + v * v
            sbuf[r, :] = sa
            sbuf[_RPW + r, :] = sb
            return 0

        lax.fori_loop(0, _RPW, row2, 0)

        for d in groups[2]:
            d.wait()

        def row3(r, acc):
            for ch in range(_NCHUNK):
                cc, co = halves(cbuf, 5, r, ch)
                dc, do = halves(cbuf, 6, r, ch)
                rr = rbuf[0, r, pl.ds(ch * 16, 16)]
                u = jnp.maximum(jnp.abs(cc + rr - dc) + co - do, 0.0)
                acc = acc + u * u
            return acc

        acc3 = lax.fori_loop(0, _RPW, row3, zero)

        for d in groups[3]:
            d.wait()

        def row4(r, acc):
            for ch in range(_NCHUNK):
                cc, co = halves(cbuf, 7, r, ch)
                dc, do = halves(cbuf, 8, r, ch)
                rr = rbuf[1, r, pl.ds(ch * 16, 16)]
                u = jnp.maximum(jnp.abs(cc - rr - dc) + co - do, 0.0)
                acc = acc + u * u
            return acc

        acc4 = lax.fori_loop(0, _RPW, row4, zero)

        for d in groups[4]:
            d.wait()

        def rowd(r, _):
            sd = zero
            for ch in range(_NCHUNK):
                cc, co = halves(cbuf, 9, r, ch)
                dc, do = halves(cbuf, 10, r, ch)
                u = jnp.maximum(jnp.abs(cc - dc) - co - do, 0.0)
                sd = sd + u * u
            sbuf[2 * _RPW + r, :] = sd
            return 0

        lax.fori_loop(0, _RPW, rowd, 0)

        for d in groups[5]:
            d.wait()

        def rown(r, _):
            sn = zero
            for ch in range(_NCHUNK):
                cc, co = halves(cbuf, 11, r, ch)
                dc, do = halves(cbuf, 12, r, ch)
                rr = rbuf[2, r, pl.ds(ch * 16, 16)]
                u = jnp.maximum(jnp.abs(cc + rr - dc) - co - do, 0.0)
                sn = sn + u * u
            sbuf[3 * _RPW + r, :] = sn
            return 0

        lax.fori_loop(0, _RPW, rown, 0)

        sbuf[4 * _RPW, :] = acc1
        sbuf[4 * _RPW + 1, :] = acc3
        sbuf[4 * _RPW + 2, :] = acc4
        sbuf[4 * _RPW + 3, :] = zero
        pltpu.sync_copy(sbuf, out_hbm.at[wid])

    return sc_k


def _combine_body(x_ref, o_ref):
    x = x_ref[...]
    inv_b = 1.0 / _BATCH
    sa = jnp.sum(x[:, 0:_RPW, :], axis=2)
    sb = jnp.sum(x[:, _RPW:2 * _RPW, :], axis=2)
    sd = jnp.sum(x[:, 2 * _RPW:3 * _RPW, :], axis=2)
    sn = jnp.sum(x[:, 3 * _RPW:4 * _RPW, :], axis=2)
    p0 = jnp.sum(x[:, 4 * _RPW, :])
    p5 = jnp.sum(x[:, 4 * _RPW + 1, :])
    p6 = jnp.sum(x[:, 4 * _RPW + 2, :])
    a = jnp.sqrt(sa)
    b = jnp.sqrt(sb)
    p1 = jnp.sum(a)
    p2 = jnp.sum(sa)
    p3 = jnp.sum(b)
    p4 = jnp.sum(sb)
    p7 = jnp.sum(jnp.maximum(2.0 - jnp.sqrt(sd), 0.0) ** 2)
    p8 = jnp.sum(jnp.sqrt(sn))
    p9 = jnp.sum(sn)
    loss = (p0 * inv_b
            + p2 * inv_b + 2.0 * (p1 * inv_b) * (p3 * inv_b) + p4 * inv_b
            + p7 * inv_b
            + p5 * inv_b + p6 * inv_b
            + 4.0 - 4.0 * p8 * inv_b + p9 * inv_b)
    o_ref[0, 0] = loss


def _tc_combine(partials):
    return pl.pallas_call(
        _combine_body,
        out_shape=jax.ShapeDtypeStruct((1, 1), jnp.float32),
        in_specs=[pl.BlockSpec(memory_space=pltpu.VMEM)],
        out_specs=pl.BlockSpec(memory_space=pltpu.SMEM),
    )(partials)


def kernel(nf1, nf2, nf3, nf4, disjoint, nf3_neg, class_emb, rel_emb):
    b = _BATCH
    cols = jnp.concatenate(
        [nf1[:b], nf2[:b], nf3[:b], nf4[:b], disjoint[:b], nf3_neg[:b]], axis=1)
    idx_w = cols.reshape(_NWORKERS, _RPW, 16).transpose(0, 2, 1)
    partials = _sc_stage()(idx_w, class_emb, rel_emb)
    return _tc_combine(partials)[0, 0]
